# baseline (speedup 1.0000x reference)
import functools

import jax
import jax.numpy as jnp
from jax import lax
from jax.experimental import pallas as pl
from jax.experimental.pallas import tpu as pltpu
from jax.experimental.pallas import tpu_sc as plsc

BATCH = 16384
D = 128
NC = 2
NS = 16
NW = NC * NS
CHUNK = 128
S = 1
H = BATCH // S
ROWS_PER_W = H // NW
CHUNKS_PER_W = ROWS_PER_W // CHUNK
NT = 2 * CHUNKS_PER_W
NB = min(NT, 7)
BM = 8192


def _sc_gather(user_idx, item_idx, user_table, item_table):
    mesh = plsc.VectorSubcoreMesh(
        core_axis_name="c", subcore_axis_name="s", num_cores=NC, num_subcores=NS
    )

    @functools.partial(
        pl.kernel,
        mesh=mesh,
        out_type=(
            jax.ShapeDtypeStruct((H, D), jnp.float32),
            jax.ShapeDtypeStruct((H, D), jnp.float32),
        ),
        scratch_types=[
            pltpu.VMEM((ROWS_PER_W,), jnp.int32),
            pltpu.VMEM((ROWS_PER_W,), jnp.int32),
            pltpu.VMEM((NB, CHUNK, D), jnp.float32),
            pltpu.SemaphoreType.DMA((NB,)),
            pltpu.SemaphoreType.DMA((NB,)),
            pltpu.SemaphoreType.DMA((2,)),
        ],
    )
    def gather_kernel(ui_hbm, ii_hbm, ut_hbm, it_hbm, u_out, i_out,
                      idx_u, idx_i, bufs, gsem, wsem, isem):
        wid = lax.axis_index("s") * NC + lax.axis_index("c")
        base = wid * CHUNKS_PER_W
        hu = pltpu.async_copy(
            ui_hbm.at[pl.ds(base * CHUNK, ROWS_PER_W)], idx_u, isem.at[0])
        hi = pltpu.async_copy(
            ii_hbm.at[pl.ds(base * CHUNK, ROWS_PER_W)], idx_i, isem.at[1])
        hu.wait()

        def chunk(t):
            j = t % CHUNKS_PER_W
            if t < CHUNKS_PER_W:
                return ut_hbm, idx_u.at[pl.ds(j * CHUNK, CHUNK)], u_out, j
            return it_hbm, idx_i.at[pl.ds(j * CHUNK, CHUNK)], i_out, j

        GLEAD = 5
        ghandles = [None] * NT
        whandles = [None] * NT

        def retire(tt):
            mm = tt % NB
            _, _, out2, j2 = chunk(tt)
            ghandles[tt].wait()
            whandles[tt] = pltpu.async_copy(
                bufs.at[mm], out2.at[pl.ds((base + j2) * CHUNK, CHUNK)], wsem.at[mm]
            )

        for t in range(NT):
            m = t % NB
            if t == CHUNKS_PER_W:
                hi.wait()
            if t >= NB:
                whandles[t - NB].wait()
            table, idxref, _, _ = chunk(t)
            ghandles[t] = pltpu.async_copy(table.at[idxref], bufs.at[m], gsem.at[m])
            if t >= GLEAD:
                retire(t - GLEAD)
        for tt in range(NT - GLEAD, NT):
            retire(tt)
        for tt in range(max(0, NT - NB), NT):
            whandles[tt].wait()

    return gather_kernel(user_idx, item_idx, user_table, item_table)


def _tc_mlp_slice(u_rows, i_rows, W1, b2, y_prev, s):
    off = s * (H // BM)

    def body(u_ref, i_ref, w_ref, b_ref, *rest):
        o_ref = rest[-1]
        dn = (((1,), (1,)), ((), ()))
        acc = lax.dot_general(u_ref[...], w_ref[:, 0:D], dn,
                              preferred_element_type=jnp.float32)
        acc += lax.dot_general(i_ref[...], w_ref[:, D : 2 * D], dn,
                               preferred_element_type=jnp.float32)
        acc += b_ref[...]
        o_ref[...] = jnp.maximum(acc, 0.0)

    in_specs = [
        pl.BlockSpec((BM, D), lambda i: (i, 0)),
        pl.BlockSpec((BM, D), lambda i: (i, 0)),
        pl.BlockSpec((D, 2 * D), lambda i: (0, 0)),
        pl.BlockSpec((1, D), lambda i: (0, 0)),
    ]
    args = [u_rows, i_rows, W1, b2]
    aliases = {}
    if y_prev is not None:
        in_specs.append(pl.BlockSpec(memory_space=pl.ANY))
        args.append(y_prev)
        aliases = {4: 0}

    return pl.pallas_call(
        body,
        grid=(H // BM,),
        in_specs=in_specs,
        out_specs=pl.BlockSpec((BM, D), lambda i: (i + off, 0)),
        out_shape=jax.ShapeDtypeStruct((BATCH, D), jnp.float32),
        input_output_aliases=aliases,
    )(*args)


def kernel(user_indices, item_indices, user_table, item_table, W1, b1):
    b2 = b1.reshape(1, D)
    gathered = []
    for s in range(S):
        ui = lax.slice(user_indices, (s * H,), ((s + 1) * H,))
        ii = lax.slice(item_indices, (s * H,), ((s + 1) * H,))
        gathered.append(_sc_gather(ui, ii, user_table, item_table))
    y = None
    for s in range(S):
        u_rows, i_rows = gathered[s]
        y = _tc_mlp_slice(u_rows, i_rows, W1, b2, y, s)
    return y


# speedup vs baseline: 1.0086x; 1.0086x over previous
import functools

import jax
import jax.numpy as jnp
from jax import lax
from jax.experimental import pallas as pl
from jax.experimental.pallas import tpu as pltpu
from jax.experimental.pallas import tpu_sc as plsc

BATCH = 16384
D = 128
NC = 2
NS = 16
NW = NC * NS
CHUNK = 128
S = 1
H = BATCH // S
ROWS_PER_W = H // NW
CHUNKS_PER_W = ROWS_PER_W // CHUNK
NT = 2 * CHUNKS_PER_W
NB = min(NT, 7)
BM = 8192


def _sc_gather(user_idx, item_idx, user_table, item_table):
    mesh = plsc.VectorSubcoreMesh(
        core_axis_name="c", subcore_axis_name="s", num_cores=NC, num_subcores=NS
    )

    @functools.partial(
        pl.kernel,
        mesh=mesh,
        out_type=(
            jax.ShapeDtypeStruct((H, D), jnp.float32),
            jax.ShapeDtypeStruct((H, D), jnp.float32),
        ),
        scratch_types=[
            pltpu.VMEM((ROWS_PER_W,), jnp.int32),
            pltpu.VMEM((ROWS_PER_W,), jnp.int32),
            pltpu.VMEM((NB, CHUNK, D), jnp.float32),
            pltpu.SemaphoreType.DMA((NB,)),
            pltpu.SemaphoreType.DMA((NB,)),
            pltpu.SemaphoreType.DMA((2,)),
        ],
    )
    def gather_kernel(ui_hbm, ii_hbm, ut_hbm, it_hbm, u_out, i_out,
                      idx_u, idx_i, bufs, gsem, wsem, isem):
        wid = lax.axis_index("s") * NC + lax.axis_index("c")
        base = wid * CHUNKS_PER_W
        hu = pltpu.async_copy(
            ui_hbm.at[pl.ds(base * CHUNK, ROWS_PER_W)], idx_u, isem.at[0])
        hi = pltpu.async_copy(
            ii_hbm.at[pl.ds(base * CHUNK, ROWS_PER_W)], idx_i, isem.at[1])
        hu.wait()

        def chunk(t):
            j = t % CHUNKS_PER_W
            if t < CHUNKS_PER_W:
                return ut_hbm, idx_u.at[pl.ds(j * CHUNK, CHUNK)], u_out, j
            return it_hbm, idx_i.at[pl.ds(j * CHUNK, CHUNK)], i_out, j

        GLEAD = 4
        ghandles = [None] * NT
        whandles = [None] * NT

        def retire(tt):
            mm = tt % NB
            _, _, out2, j2 = chunk(tt)
            ghandles[tt].wait()
            whandles[tt] = pltpu.async_copy(
                bufs.at[mm], out2.at[pl.ds((base + j2) * CHUNK, CHUNK)], wsem.at[mm]
            )

        for t in range(NT):
            m = t % NB
            if t == CHUNKS_PER_W:
                hi.wait()
            if t >= NB:
                whandles[t - NB].wait()
            table, idxref, _, _ = chunk(t)
            ghandles[t] = pltpu.async_copy(table.at[idxref], bufs.at[m], gsem.at[m])
            if t >= GLEAD:
                retire(t - GLEAD)
        for tt in range(NT - GLEAD, NT):
            retire(tt)
        for tt in range(max(0, NT - NB), NT):
            whandles[tt].wait()

    return gather_kernel(user_idx, item_idx, user_table, item_table)


def _tc_mlp_slice(u_rows, i_rows, W1, b2, y_prev, s):
    off = s * (H // BM)

    def body(u_ref, i_ref, w_ref, b_ref, *rest):
        o_ref = rest[-1]
        dn = (((1,), (1,)), ((), ()))
        acc = lax.dot_general(u_ref[...], w_ref[:, 0:D], dn,
                              preferred_element_type=jnp.float32)
        acc += lax.dot_general(i_ref[...], w_ref[:, D : 2 * D], dn,
                               preferred_element_type=jnp.float32)
        acc += b_ref[...]
        o_ref[...] = jnp.maximum(acc, 0.0)

    in_specs = [
        pl.BlockSpec((BM, D), lambda i: (i, 0)),
        pl.BlockSpec((BM, D), lambda i: (i, 0)),
        pl.BlockSpec((D, 2 * D), lambda i: (0, 0)),
        pl.BlockSpec((1, D), lambda i: (0, 0)),
    ]
    args = [u_rows, i_rows, W1, b2]
    aliases = {}
    if y_prev is not None:
        in_specs.append(pl.BlockSpec(memory_space=pl.ANY))
        args.append(y_prev)
        aliases = {4: 0}

    return pl.pallas_call(
        body,
        grid=(H // BM,),
        in_specs=in_specs,
        out_specs=pl.BlockSpec((BM, D), lambda i: (i + off, 0)),
        out_shape=jax.ShapeDtypeStruct((BATCH, D), jnp.float32),
        input_output_aliases=aliases,
    )(*args)


def kernel(user_indices, item_indices, user_table, item_table, W1, b1):
    b2 = b1.reshape(1, D)
    gathered = []
    for s in range(S):
        ui = lax.slice(user_indices, (s * H,), ((s + 1) * H,))
        ii = lax.slice(item_indices, (s * H,), ((s + 1) * H,))
        gathered.append(_sc_gather(ui, ii, user_table, item_table))
    y = None
    for s in range(S):
        u_rows, i_rows = gathered[s]
        y = _tc_mlp_slice(u_rows, i_rows, W1, b2, y, s)
    return y
